# baseline (speedup 1.0000x reference)
import functools
import jax
import jax.numpy as jnp
from jax import lax
from jax.experimental import pallas as pl
from jax.experimental.pallas import tpu as pltpu
from jax.experimental.pallas import tpu_sc as plsc

_EPS = 1e-8
_NW = 32


def _sc_params(mus, log_scales, vids, vhb):
    nn = mus.shape[0]
    cpw = nn // _NW
    mesh = plsc.VectorSubcoreMesh(core_axis_name="c", subcore_axis_name="s",
                                  num_cores=2, num_subcores=16)

    @functools.partial(
        pl.kernel,
        out_type=[
            jax.ShapeDtypeStruct((nn,), jnp.float32),
            jax.ShapeDtypeStruct((nn,), jnp.float32),
            jax.ShapeDtypeStruct((nn,), jnp.float32),
        ],
        mesh=mesh,
        scratch_types=[
            pltpu.VMEM((cpw,), jnp.float32),
            pltpu.VMEM((cpw,), jnp.float32),
            pltpu.VMEM((cpw,), jnp.int32),
            pltpu.VMEM((cpw,), jnp.float32),
            pltpu.VMEM((cpw,), jnp.float32),
            pltpu.VMEM((cpw,), jnp.float32),
            pltpu.VMEM((cpw,), jnp.float32),
            pltpu.SemaphoreType.DMA,
        ],
    )
    def k(mus_hbm, ls_hbm, vids_hbm, vhb_hbm, isch_hbm, muh_hbm, dh_hbm,
          mus_v, ls_v, vids_v, hb_v, o1_v, o2_v, o3_v, sem):
        wid = lax.axis_index("s") * 2 + lax.axis_index("c")
        base = wid * cpw
        pltpu.sync_copy(mus_hbm.at[pl.ds(base, cpw)], mus_v)
        pltpu.sync_copy(ls_hbm.at[pl.ds(base, cpw)], ls_v)
        pltpu.sync_copy(vids_hbm.at[pl.ds(base, cpw)], vids_v)
        pltpu.async_copy(vhb_hbm.at[vids_v], hb_v, sem).wait()

        def body(i, carry):
            sl = pl.ds(i * 16, 16)
            mu = mus_v[sl]
            ls = jnp.maximum(ls_v[sl], -5.0)
            hb = hb_v[sl]
            isch = jnp.exp(-ls) * 0.5
            o1_v[sl] = isch
            o2_v[sl] = (mu + hb) * isch
            o3_v[sl] = (2.0 * hb) * isch
            return carry

        lax.fori_loop(0, cpw // 16, body, 0)
        pltpu.sync_copy(o1_v, isch_hbm.at[pl.ds(base, cpw)])
        pltpu.sync_copy(o2_v, muh_hbm.at[pl.ds(base, cpw)])
        pltpu.sync_copy(o3_v, dh_hbm.at[pl.ds(base, cpw)])

    return k(mus, log_scales, vids, vhb)


def _tc_body(data_ref, isch_ref, muh_ref, dh_ref, vlow_ref, vhigh_ref,
             out_ref):
    npv, b = out_ref.shape
    v = pl.program_id(0)
    low = vlow_ref[v, 0]
    high = vhigh_ref[v, 0]

    sd_row = (data_ref[...].reshape(1, b) - low) * (1.0 / (high - low))
    low_m = sd_row < 0.01
    high_m = sd_row > 0.99
    ar = jnp.where(low_m, 0.0, jnp.where(high_m, -0.5, 0.5 * (1.0 + _EPS)))
    al = jnp.where(low_m, 0.5, jnp.where(high_m, 0.0, -0.5))
    beta = jnp.where(low_m, 0.5, jnp.where(high_m, 0.5 + _EPS, 0.5 * _EPS))

    def col(x):
        return x.reshape(npv, 1)

    isch_c = col(isch_ref[...].reshape(1, npv))
    muh_c = col(muh_ref[...].reshape(1, npv))
    dh_c = col(dh_ref[...].reshape(1, npv))

    argl = sd_row * isch_c - muh_c
    tl = jnp.tanh(argl)
    tr = jnp.tanh(argl + dh_c)
    numer = ar * tr + al * tl + beta
    out_ref[...] = jnp.log(numer)


def kernel(data, node_mars, mus, log_scales, vids, d2vids, vrangeslow,
           vrangeshigh, vhbinsizes):
    nv, b = data.shape
    nn = mus.shape[0]
    npv = nn // nv
    isch, muh, dh = _sc_params(mus, log_scales, vids, vhbinsizes.reshape(nv))
    return pl.pallas_call(
        _tc_body,
        grid=(nv,),
        in_specs=[
            pl.BlockSpec((1, 1, b), lambda v: (v, 0, 0)),
            pl.BlockSpec((1, 1, npv), lambda v: (v, 0, 0)),
            pl.BlockSpec((1, 1, npv), lambda v: (v, 0, 0)),
            pl.BlockSpec((1, 1, npv), lambda v: (v, 0, 0)),
            pl.BlockSpec(memory_space=pltpu.SMEM),
            pl.BlockSpec(memory_space=pltpu.SMEM),
        ],
        out_specs=pl.BlockSpec((npv, b), lambda v: (v, 0)),
        out_shape=jax.ShapeDtypeStruct((nn, b), jnp.float32),
    )(data.reshape(nv, 1, b), isch.reshape(nv, 1, npv),
      muh.reshape(nv, 1, npv), dh.reshape(nv, 1, npv),
      vrangeslow, vrangeshigh)


# speedup vs baseline: 3.0237x; 3.0237x over previous
import jax
import jax.numpy as jnp
from jax.experimental import pallas as pl
from jax.experimental.pallas import tpu as pltpu

_EPS = 1e-8


def _tc_body(data_ref, mus_ref, ls_ref, vlow_ref, vhigh_ref, vhb_ref,
             out_ref):
    npv, b = out_ref.shape
    v = pl.program_id(0)
    low = vlow_ref[v, 0]
    high = vhigh_ref[v, 0]
    hb = vhb_ref[v, 0]

    sd_row = (data_ref[...].reshape(1, b) - low) * (1.0 / (high - low))
    low_m = sd_row < 0.01
    high_m = sd_row > 0.99
    ar = jnp.where(low_m, 0.0, jnp.where(high_m, -0.5, 0.5 * (1.0 + _EPS)))
    al = jnp.where(low_m, 0.5, jnp.where(high_m, 0.0, -0.5))
    beta = jnp.where(low_m, 0.5, jnp.where(high_m, 0.5 + _EPS, 0.5 * _EPS))

    mu_row = mus_ref[...].reshape(1, npv)
    ls_row = jnp.maximum(ls_ref[...].reshape(1, npv), -5.0)
    isch_row = jnp.exp(-ls_row) * 0.5
    muh_row = (mu_row + hb) * isch_row
    dh_row = (2.0 * hb) * isch_row

    def col(x):
        return x.reshape(npv, 1)

    isch_c = col(isch_row)
    muh_c = col(muh_row)
    dh_c = col(dh_row)

    argl = sd_row * isch_c - muh_c
    tl = jnp.tanh(argl)
    tr = jnp.tanh(argl + dh_c)
    numer = ar * tr + al * tl + beta
    out_ref[...] = jnp.log(numer)


def kernel(data, node_mars, mus, log_scales, vids, d2vids, vrangeslow,
           vrangeshigh, vhbinsizes):
    nv, b = data.shape
    nn = mus.shape[0]
    npv = nn // nv
    return pl.pallas_call(
        _tc_body,
        grid=(nv,),
        in_specs=[
            pl.BlockSpec((1, 1, b), lambda v: (v, 0, 0)),
            pl.BlockSpec((1, 1, npv), lambda v: (v, 0, 0)),
            pl.BlockSpec((1, 1, npv), lambda v: (v, 0, 0)),
            pl.BlockSpec(memory_space=pltpu.SMEM),
            pl.BlockSpec(memory_space=pltpu.SMEM),
            pl.BlockSpec(memory_space=pltpu.SMEM),
        ],
        out_specs=pl.BlockSpec((npv, b), lambda v: (v, 0)),
        out_shape=jax.ShapeDtypeStruct((nn, b), jnp.float32),
    )(data.reshape(nv, 1, b), mus.reshape(nv, 1, npv),
      log_scales.reshape(nv, 1, npv), vrangeslow, vrangeshigh, vhbinsizes)


# speedup vs baseline: 3.4970x; 1.1565x over previous
import jax
import jax.numpy as jnp
from jax.experimental import pallas as pl
from jax.experimental.pallas import tpu as pltpu

_EPS = 1e-8
_VPB = 4


def _tc_body(data_ref, mus_ref, ls_ref, vlow_ref, vhigh_ref, vhb_ref,
             out_ref):
    nrows, b = out_ref.shape
    npv = nrows // _VPB
    v0 = pl.program_id(0) * _VPB

    for h in range(_VPB):
        low = vlow_ref[v0 + h, 0]
        high = vhigh_ref[v0 + h, 0]
        hb = vhb_ref[v0 + h, 0]

        sd_row = (data_ref[0, h, :].reshape(1, b) - low) * (1.0 / (high - low))
        low_m = sd_row < 0.01
        high_m = sd_row > 0.99
        ar = jnp.where(low_m, 0.0, jnp.where(high_m, -0.5, 0.5 * (1.0 + _EPS)))
        al = jnp.where(low_m, 0.5, jnp.where(high_m, 0.0, -0.5))
        beta = jnp.where(low_m, 0.5, jnp.where(high_m, 0.5 + _EPS, 0.5 * _EPS))

        mu_row = mus_ref[0, h, :].reshape(1, npv)
        ls_row = jnp.maximum(ls_ref[0, h, :].reshape(1, npv), -5.0)
        isch_row = jnp.exp(-ls_row) * 0.5
        muh_row = (mu_row + hb) * isch_row
        dh_row = (2.0 * hb) * isch_row

        isch_c = isch_row.reshape(npv, 1)
        muh_c = muh_row.reshape(npv, 1)
        dh_c = dh_row.reshape(npv, 1)

        argl = sd_row * isch_c - muh_c
        tl = jnp.tanh(argl)
        tr = jnp.tanh(argl + dh_c)
        numer = ar * tr + al * tl + beta
        out_ref[h * npv:(h + 1) * npv, :] = jnp.log(numer)


def kernel(data, node_mars, mus, log_scales, vids, d2vids, vrangeslow,
           vrangeshigh, vhbinsizes):
    nv, b = data.shape
    nn = mus.shape[0]
    npv = nn // nv
    ng = nv // _VPB
    return pl.pallas_call(
        _tc_body,
        grid=(ng,),
        in_specs=[
            pl.BlockSpec((1, _VPB, b), lambda v: (v, 0, 0)),
            pl.BlockSpec((1, _VPB, npv), lambda v: (v, 0, 0)),
            pl.BlockSpec((1, _VPB, npv), lambda v: (v, 0, 0)),
            pl.BlockSpec(memory_space=pltpu.SMEM),
            pl.BlockSpec(memory_space=pltpu.SMEM),
            pl.BlockSpec(memory_space=pltpu.SMEM),
        ],
        out_specs=pl.BlockSpec((_VPB * npv, b), lambda v: (v, 0)),
        out_shape=jax.ShapeDtypeStruct((nn, b), jnp.float32),
    )(data.reshape(ng, _VPB, b), mus.reshape(ng, _VPB, npv),
      log_scales.reshape(ng, _VPB, npv), vrangeslow, vrangeshigh, vhbinsizes)
